# Initial kernel scaffold; baseline (speedup 1.0000x reference)
import jax
import jax.numpy as jnp
from jax.experimental import pallas as pl


def kernel(x, batch, ei, W):
    raise NotImplementedError("write your pallas kernel here")


# speedup vs baseline: 35.9625x; 35.9625x over previous
import functools

import jax
import jax.numpy as jnp
from jax import lax
from jax.experimental import pallas as pl
from jax.experimental.pallas import tpu as pltpu
from jax.experimental.pallas import tpu_sc as plsc

H = 8
NC = 2
NS = 16
NW = NC * NS
B = 80
FB = 16


def _proj_body(x_ref, w_ref, q_ref, k_ref, v_ref, *, fqk, scaling):
    proj = lax.dot_general(x_ref[...], w_ref[...],
                           (((1,), (1,)), ((), ())),
                           preferred_element_type=jnp.float32)
    q_ref[...] = proj[:, :fqk] * scaling
    k_ref[...] = proj[:, fqk:2 * fqk]
    v_ref[...] = proj[:, 2 * fqk:]


def _project(x, w, fqk, fv, scaling):
    n, fin = x.shape
    tot = w.shape[0]
    bn = 1000 if n % 1000 == 0 else n
    f32 = jnp.float32
    return pl.pallas_call(
        functools.partial(_proj_body, fqk=fqk, scaling=scaling),
        grid=(n // bn,),
        in_specs=[
            pl.BlockSpec((bn, fin), lambda i: (i, 0)),
            pl.BlockSpec((tot, fin), lambda i: (0, 0)),
        ],
        out_specs=[
            pl.BlockSpec((bn, fqk), lambda i: (i, 0)),
            pl.BlockSpec((bn, fqk), lambda i: (i, 0)),
            pl.BlockSpec((bn, fv), lambda i: (i, 0)),
        ],
        out_shape=[
            jax.ShapeDtypeStruct((n, fqk), f32),
            jax.ShapeDtypeStruct((n, fqk), f32),
            jax.ShapeDtypeStruct((n, fv), f32),
        ],
    )(x, w)


def _sc_body(q_hbm, k_hbm, v_hbm, src_hbm, dest_hbm, zinit_hbm, out_hbm,
             src_v, dest_v, q_rows, k_rows, v_rows, flush_buf,
             flush_idx, acc_sh, sem_q, sem_k, sem_v,
             *, ew, nblk, roww, npad, dummy):
    cid = lax.axis_index("c")
    sid = lax.axis_index("s")
    wid = sid * NC + cid
    sl = npad // NS

    pltpu.sync_copy(zinit_hbm, acc_sh.at[pl.ds(sid * sl, sl)])
    plsc.subcore_barrier()

    iota = lax.iota(jnp.int32, 16)
    lanes_lt8 = iota < 8
    dummy_vec = jnp.full((16,), dummy, jnp.int32)
    zero16 = jnp.zeros((16,), jnp.float32)

    flush_idx[...] = dummy_vec

    def flush(cur, count, accs):
        for j in range(8):
            flush_buf[count, pl.ds(j * 16, 16)] = accs[j]
        flush_buf[count, pl.ds(8 * 16, 16)] = accs[8]
        idxv = flush_idx[...]
        flush_idx[...] = jnp.where(iota == count,
                                   jnp.where(cur < 0, dummy, cur), idxv)
        count = count + 1

        @pl.when(count == FB)
        def _():
            pltpu.sync_copy(flush_buf, acc_sh.at[flush_idx], add=True)
            flush_idx[...] = dummy_vec

        return jnp.where(count == FB, 0, count)

    base = wid * ew

    def block_body(blk, carry):
        off = pl.multiple_of(base + blk * B, 8)
        pltpu.sync_copy(src_hbm.at[pl.ds(off, B)], src_v)
        pltpu.sync_copy(dest_hbm.at[pl.ds(off, B)], dest_v)
        cq = pltpu.async_copy(q_hbm.at[src_v], q_rows, sem_q)
        ck = pltpu.async_copy(k_hbm.at[dest_v], k_rows, sem_k)
        cv = pltpu.async_copy(v_hbm.at[dest_v], v_rows, sem_v)
        cq.wait()
        ck.wait()
        cv.wait()

        def group_body(g, gcarry):
            src_vec = src_v[pl.ds(g * 16, 16)]
            carry_in = gcarry
            for j in range(16):
                cur, count, accs = carry_in
                i = g * 16 + j
                s_e = src_vec[j]
                is_new = s_e != cur
                count = lax.cond(
                    is_new,
                    lambda c, cr=cur, ac=accs: flush(cr, c, ac),
                    lambda c: c,
                    count)
                keep = jnp.where(is_new, 0.0, 1.0)
                accs = tuple(a * keep for a in accs)
                aw = zero16
                for h in range(H):
                    qs = q_rows[i, pl.ds(h * 16, 16)]
                    ks = k_rows[i, pl.ds(h * 16, 16)]
                    aw = jnp.where(iota == h, jnp.sum(qs * ks), aw)
                ex = jnp.where(lanes_lt8, jnp.exp(aw) + 1e-8, 0.0)
                new_accs = []
                for h in range(H):
                    vs = v_rows[i, pl.ds(h * 16, 16)]
                    new_accs.append(accs[h] + ex[h] * vs)
                new_accs.append(accs[8] + ex)
                carry_in = (s_e, count, tuple(new_accs))
            return carry_in

        return lax.fori_loop(0, B // 16, group_body, carry)

    carry0 = (jnp.int32(-1), jnp.int32(0), tuple(zero16 for _ in range(9)))
    cur, count, accs = lax.fori_loop(0, nblk, block_body, carry0)

    flush(cur, count, accs)
    pltpu.sync_copy(flush_buf, acc_sh.at[flush_idx], add=True)

    plsc.subcore_barrier()
    pltpu.sync_copy(acc_sh.at[pl.ds(sid * sl, sl)],
                    out_hbm.at[cid, pl.ds(sid * sl, sl)])


def _sc_attend(q, k, v, src, dest):
    n = q.shape[0]
    e = src.shape[0]
    roww = 144
    ew = e // NW
    nblk = ew // B
    npad = NS * 8 * -(-(n + 1) // (NS * 8))
    sl = npad // NS
    f32 = jnp.float32
    zinit = jnp.zeros((sl, roww), f32)

    body = functools.partial(_sc_body, ew=ew, nblk=nblk, roww=roww,
                             npad=npad, dummy=n)
    mesh = plsc.VectorSubcoreMesh(core_axis_name="c", subcore_axis_name="s",
                                  num_cores=NC, num_subcores=NS)
    return pl.kernel(
        body,
        out_type=jax.ShapeDtypeStruct((NC, npad, roww), f32),
        mesh=mesh,
        compiler_params=pltpu.CompilerParams(
            needs_layout_passes=False, use_tc_tiling_on_sc=False),
        scratch_types=[
            pltpu.VMEM((B,), jnp.int32),
            pltpu.VMEM((B,), jnp.int32),
            pltpu.VMEM((B, 128), f32),
            pltpu.VMEM((B, 128), f32),
            pltpu.VMEM((B, 128), f32),
            pltpu.VMEM((FB, roww), f32),
            pltpu.VMEM((FB,), jnp.int32),
            pltpu.VMEM_SHARED((npad, roww), f32),
            pltpu.SemaphoreType.DMA,
            pltpu.SemaphoreType.DMA,
            pltpu.SemaphoreType.DMA,
        ],
    )(q, k, v, src, dest, zinit)


def _norm_body(p_ref, out_ref):
    t = p_ref[0] + p_ref[1]
    for h in range(H):
        a = t[:, h * 16:(h + 1) * 16]
        s = t[:, 128 + h:129 + h]
        out_ref[:, h * 16:(h + 1) * 16] = jnp.where(s > 0, a / s, 0.0)


def _normalize(partial, n, fv):
    npad, roww = partial.shape[1], partial.shape[2]
    bn = 1000 if n % 1000 == 0 else n
    return pl.pallas_call(
        _norm_body,
        grid=(n // bn,),
        in_specs=[pl.BlockSpec((NC, bn, roww), lambda i: (0, i, 0))],
        out_specs=pl.BlockSpec((bn, fv), lambda i: (i, 0)),
        out_shape=jax.ShapeDtypeStruct((n, fv), jnp.float32),
    )(partial)


def kernel(x, batch, ei, W):
    n = x.shape[0]
    tot = W.shape[0]
    fqk = tot // 3
    fv = tot - 2 * fqk
    scaling = float(fqk // H) ** (-0.5)
    src = ei[0]
    dest = ei[1]
    q, k, v = _project(x, W, fqk, fv, scaling)
    partial = _sc_attend(q, k, v, src, dest)
    return _normalize(partial, n, fv)
